# baseline (speedup 1.0000x reference)
import jax
import jax.numpy as jnp
from jax.experimental import pallas as pl
from jax.experimental.pallas import tpu as pltpu

B, N, M, K = 16, 2048, 2048, 3


def _chamfer_kernel(x1_ref, x2_ref, out_ref, s1_ref, s2_ref):
    b = pl.program_id(0)

    x1 = x1_ref[0]
    x2t = jnp.transpose(x2_ref[0], (1, 0))

    a0 = x1[:, 0:1]
    a1 = x1[:, 1:2]
    a2 = x1[:, 2:3]
    b0 = x2t[0:1, :]
    b1 = x2t[1:2, :]
    b2 = x2t[2:3, :]

    d0 = a0 - b0
    dist = d0 * d0
    d1 = a1 - b1
    dist = d1 * d1 + dist
    d2 = a2 - b2
    dist = d2 * d2 + dist

    row_min = jnp.min(dist, axis=1, keepdims=True)
    col_min = jnp.min(dist, axis=0, keepdims=True)

    @pl.when(b == 0)
    def _init():
        s1_ref[0] = 0.0
        s2_ref[0] = 0.0

    s1_ref[0] += jnp.sum(row_min)
    s2_ref[0] += jnp.sum(col_min)

    @pl.when(b == B - 1)
    def _finish():
        out_ref[0, 0] = s1_ref[0] / (B * N) + s2_ref[0] / (B * M)


@jax.jit
def kernel(xyz1, xyz2):
    out = pl.pallas_call(
        _chamfer_kernel,
        grid=(B,),
        in_specs=[
            pl.BlockSpec((1, N, K), lambda b: (b, 0, 0)),
            pl.BlockSpec((1, M, K), lambda b: (b, 0, 0)),
        ],
        out_specs=pl.BlockSpec(
            (1, 1), lambda b: (0, 0), memory_space=pltpu.SMEM
        ),
        out_shape=jax.ShapeDtypeStruct((1, 1), jnp.float32),
        scratch_shapes=[
            pltpu.SMEM((1,), jnp.float32),
            pltpu.SMEM((1,), jnp.float32),
        ],
    )(xyz1, xyz2)
    return out[0, 0]


# speedup vs baseline: 1.0865x; 1.0865x over previous
import jax
import jax.numpy as jnp
from jax.experimental import pallas as pl
from jax.experimental.pallas import tpu as pltpu

B, N, M, K = 16, 2048, 2048, 3


def _chamfer_kernel(x1_ref, x2_ref, out_ref, s1_ref, s2_ref):
    b = pl.program_id(0)

    x1 = x1_ref[0]
    x2t = x2_ref[0]

    a0 = x1[:, 0:1]
    a1 = x1[:, 1:2]
    a2 = x1[:, 2:3]
    b0 = x2t[0:1, :]
    b1 = x2t[1:2, :]
    b2 = x2t[2:3, :]

    d0 = a0 - b0
    dist = d0 * d0
    d1 = a1 - b1
    dist = d1 * d1 + dist
    d2 = a2 - b2
    dist = d2 * d2 + dist

    row_min = jnp.min(dist, axis=1, keepdims=True)
    col_min = jnp.min(dist, axis=0, keepdims=True)

    @pl.when(b == 0)
    def _init():
        s1_ref[0] = 0.0
        s2_ref[0] = 0.0

    s1_ref[0] += jnp.sum(row_min)
    s2_ref[0] += jnp.sum(col_min)

    @pl.when(b == B - 1)
    def _finish():
        out_ref[0, 0] = s1_ref[0] / (B * N) + s2_ref[0] / (B * M)


@jax.jit
def kernel(xyz1, xyz2):
    x2t = jnp.transpose(xyz2, (0, 2, 1))
    out = pl.pallas_call(
        _chamfer_kernel,
        grid=(B,),
        in_specs=[
            pl.BlockSpec((1, N, K), lambda b: (b, 0, 0)),
            pl.BlockSpec((1, K, M), lambda b: (b, 0, 0)),
        ],
        out_specs=pl.BlockSpec(
            (1, 1), lambda b: (0, 0), memory_space=pltpu.SMEM
        ),
        out_shape=jax.ShapeDtypeStruct((1, 1), jnp.float32),
        scratch_shapes=[
            pltpu.SMEM((1,), jnp.float32),
            pltpu.SMEM((1,), jnp.float32),
        ],
    )(xyz1, x2t)
    return out[0, 0]
